# Initial kernel scaffold; baseline (speedup 1.0000x reference)
import jax
import jax.numpy as jnp
from jax.experimental import pallas as pl


def kernel(x, edge_weight, neigh_mask, W_r, b_r, dom_r, W_ri, b_ri, W_r1, b_r1, W_r2, b_r2, dom_res, W_g1, W_mu, W_ls, W_t0, b_t0, W_t1, b_t1, dom_tg, W_l1, b_l1, dom_1, W_l2, b_l2, edge_index, domain_idx):
    raise NotImplementedError("write your pallas kernel here")


# speedup vs baseline: 1.3080x; 1.3080x over previous
import functools

import jax
import jax.numpy as jnp
from jax import lax
from jax.experimental import pallas as pl
from jax.experimental.pallas import tpu as pltpu
from jax.experimental.pallas import tpu_sc as plsc

N = 4096
E = 131072
DIN = 512
DHID = 1024
DO = 256

F32 = jnp.float32
I32 = jnp.int32

NC = 2
NS = 16
NW = NC * NS

_SC_MESH = dict(core_axis_name="c", subcore_axis_name="s", num_cores=NC,
                num_subcores=NS)


def _mesh():
    return plsc.VectorSubcoreMesh(**_SC_MESH)


MBLK = 512


def _front_body(x_ref, w_ref, b_ref, y_ref):
    y_ref[...] = jnp.dot(x_ref[...], w_ref[...],
                         preferred_element_type=F32) + b_ref[...]


def _tc_front(x, w, b2):
    return pl.pallas_call(
        _front_body,
        grid=(N // MBLK,),
        in_specs=[
            pl.BlockSpec((MBLK, DIN), lambda i: (i, 0)),
            pl.BlockSpec((DIN, DHID), lambda i: (0, 0)),
            pl.BlockSpec((1, DHID), lambda i: (0, 0)),
        ],
        out_specs=pl.BlockSpec((MBLK, DHID), lambda i: (i, 0)),
        out_shape=jax.ShapeDtypeStruct((N, DHID), F32),
    )(x, w, b2)


def _mid_body(y_ref, oh_ref, domr_ref, wri_ref, bri_ref, domres_ref,
              wr1_ref, br1_ref, wr2_ref, br2_ref, wg1_ref,
              feat_ref, hw1_ref):
    oh = oh_ref[...]
    h1 = jnp.maximum(y_ref[...], 0.0) + jnp.dot(
        oh, domr_ref[...], preferred_element_type=F32)
    h2 = jnp.maximum(
        jnp.dot(h1, wri_ref[...], preferred_element_type=F32) + bri_ref[...],
        0.0) + jnp.dot(oh, domres_ref[...], preferred_element_type=F32)
    h3 = h2 + jnp.maximum(
        jnp.dot(h2, wr1_ref[...], preferred_element_type=F32) + br1_ref[...],
        0.0)
    h4 = h3 + jnp.maximum(
        jnp.dot(h3, wr2_ref[...], preferred_element_type=F32) + br2_ref[...],
        0.0)
    feat_ref[...] = h4
    hw1_ref[...] = jnp.dot(h4, wg1_ref[...], preferred_element_type=F32)


def _tc_mid(y_all, oh2, dom_r, w_ri, bri2, dom_res, w_r1, br12, w_r2, br22,
            w_g1):
    m = y_all.shape[0]
    return pl.pallas_call(
        _mid_body,
        grid=(m // MBLK,),
        in_specs=[
            pl.BlockSpec((MBLK, DHID), lambda i: (i, 0)),
            pl.BlockSpec((MBLK, 4), lambda i: (i, 0)),
            pl.BlockSpec((4, DHID), lambda i: (0, 0)),
            pl.BlockSpec((DHID, DO), lambda i: (0, 0)),
            pl.BlockSpec((1, DO), lambda i: (0, 0)),
            pl.BlockSpec((4, DO), lambda i: (0, 0)),
            pl.BlockSpec((DO, DO), lambda i: (0, 0)),
            pl.BlockSpec((1, DO), lambda i: (0, 0)),
            pl.BlockSpec((DO, DO), lambda i: (0, 0)),
            pl.BlockSpec((1, DO), lambda i: (0, 0)),
            pl.BlockSpec((DO, DO), lambda i: (0, 0)),
        ],
        out_specs=[
            pl.BlockSpec((MBLK, DO), lambda i: (i, 0)),
            pl.BlockSpec((MBLK, DO), lambda i: (i, 0)),
        ],
        out_shape=[
            jax.ShapeDtypeStruct((m, DO), F32),
            jax.ShapeDtypeStruct((m, DO), F32),
        ],
    )(y_all, oh2, dom_r, w_ri, bri2, dom_res, w_r1, br12, w_r2, br22, w_g1)


def _gcnmid_body(agg_ref, wml_ref, out_ref):
    hid = jnp.maximum(agg_ref[...], 0.0)
    out_ref[...] = jnp.dot(hid, wml_ref[...], preferred_element_type=F32)


def _tc_gcnmid(agg_all, wml):
    m = agg_all.shape[0]
    return pl.pallas_call(
        _gcnmid_body,
        grid=(m // MBLK,),
        in_specs=[
            pl.BlockSpec((MBLK, DO), lambda i: (i, 0)),
            pl.BlockSpec((DO, 2 * DO), lambda i: (0, 0)),
        ],
        out_specs=pl.BlockSpec((MBLK, 2 * DO), lambda i: (i, 0)),
        out_shape=jax.ShapeDtypeStruct((m, 2 * DO), F32),
    )(agg_all, wml)


def _tail_body(feat_ref, mu_ref, lsr_ref, eps_ref, oh_ref,
               wt0_ref, bt0_ref, wt1_ref, bt1_ref, domtg_ref,
               wl1_ref, bl1_ref, dom1_ref, wl2_ref, bl2_ref, out_ref):
    oh = oh_ref[...]
    ls = jnp.clip(lsr_ref[...], -10.0, 10.0)
    z = mu_ref[...] + eps_ref[...] * jnp.exp(ls)
    g = jnp.maximum(
        jnp.dot(z, wt0_ref[...], preferred_element_type=F32) + bt0_ref[...],
        0.0)
    g = jnp.maximum(
        jnp.dot(g, wt1_ref[...], preferred_element_type=F32) + bt1_ref[...],
        0.0) + jnp.dot(oh, domtg_ref[...], preferred_element_type=F32)
    feat = jnp.concatenate([feat_ref[...], g], axis=1)
    l1 = jnp.maximum(
        jnp.dot(feat, wl1_ref[...], preferred_element_type=F32) + bl1_ref[...],
        0.0) + jnp.dot(oh, dom1_ref[...], preferred_element_type=F32)
    out_ref[...] = jnp.maximum(
        jnp.dot(l1, wl2_ref[...], preferred_element_type=F32) + bl2_ref[...],
        0.0)


def _tc_tail(feat, mu, lsr, eps, oh2, w_t0, bt02, w_t1, bt12, dom_tg,
             w_l1, bl12, dom_1, w_l2, bl22):
    m = feat.shape[0]
    return pl.pallas_call(
        _tail_body,
        grid=(m // MBLK,),
        in_specs=[
            pl.BlockSpec((MBLK, DO), lambda i: (i, 0)),
            pl.BlockSpec((MBLK, DO), lambda i: (i, 0)),
            pl.BlockSpec((MBLK, DO), lambda i: (i, 0)),
            pl.BlockSpec((MBLK, DO), lambda i: (i, 0)),
            pl.BlockSpec((MBLK, 4), lambda i: (i, 0)),
            pl.BlockSpec((DO, DO), lambda i: (0, 0)),
            pl.BlockSpec((1, DO), lambda i: (0, 0)),
            pl.BlockSpec((DO, DO), lambda i: (0, 0)),
            pl.BlockSpec((1, DO), lambda i: (0, 0)),
            pl.BlockSpec((4, DO), lambda i: (0, 0)),
            pl.BlockSpec((2 * DO, DO), lambda i: (0, 0)),
            pl.BlockSpec((1, DO), lambda i: (0, 0)),
            pl.BlockSpec((4, DO), lambda i: (0, 0)),
            pl.BlockSpec((DO, DO), lambda i: (0, 0)),
            pl.BlockSpec((1, DO), lambda i: (0, 0)),
        ],
        out_specs=pl.BlockSpec((MBLK, DO), lambda i: (i, 0)),
        out_shape=jax.ShapeDtypeStruct((m, DO), F32),
    )(feat, mu, lsr, eps, oh2, w_t0, bt02, w_t1, bt12, dom_tg,
      w_l1, bl12, dom_1, w_l2, bl22)


def _gram_body(pxb_ref, px_ref, g_ref):
    g_ref[...] = lax.dot_general(
        pxb_ref[...], px_ref[...], (((1,), (1,)), ((), ())),
        preferred_element_type=F32)


def _tc_gram(px):
    return pl.pallas_call(
        _gram_body,
        grid=(N // MBLK,),
        in_specs=[
            pl.BlockSpec((MBLK, DO), lambda i: (i, 0)),
            pl.BlockSpec((N, DO), lambda i: (0, 0)),
        ],
        out_specs=pl.BlockSpec((MBLK, N), lambda i: (i, 0)),
        out_shape=jax.ShapeDtypeStruct((N, N), F32),
    )(px, px)


def _readout_body(nm_ref, px_ref, out_ref):
    nm = nm_ref[...]
    vsum = jnp.dot(nm, px_ref[...], preferred_element_type=F32)
    rsum = jnp.sum(nm, axis=1, keepdims=True)
    gz = vsum / rsum
    nrm = jnp.sqrt(jnp.sum(gz * gz, axis=1, keepdims=True))
    out_ref[...] = gz / jnp.maximum(nrm, 1e-12)


def _tc_readout(nm, px):
    return pl.pallas_call(
        _readout_body,
        grid=(N // MBLK,),
        in_specs=[
            pl.BlockSpec((MBLK, N), lambda i: (i, 0)),
            pl.BlockSpec((N, DO), lambda i: (0, 0)),
        ],
        out_specs=pl.BlockSpec((MBLK, DO), lambda i: (i, 0)),
        out_shape=jax.ShapeDtypeStruct((N, DO), F32),
    )(nm, px)


def _loss_body(mu_ref, lsr_ref, ps_ref, ns_ref, out_ref):
    mu = mu_ref[...]
    ls = jnp.clip(lsr_ref[...], -10.0, 10.0)
    t = 1.0 + 2.0 * ls - mu * mu - jnp.exp(2.0 * ls)
    kl = -0.5 * (jnp.sum(t) / N)
    ps = ps_ref[...]
    ns = ns_ref[...]
    pterm = jnp.log(jnp.maximum(jax.nn.sigmoid(ps), 1e-15))
    nterm = jnp.log(jnp.maximum(1.0 - jax.nn.sigmoid(ns), 1e-15))
    recon = -(jnp.sum(pterm) / E) - (jnp.sum(nterm) / E)
    out_ref[...] = jnp.reshape(kl / N + recon, (1, 1))


def _tc_loss(mu, lsr, ps2, ns2):
    return pl.pallas_call(
        _loss_body,
        grid=(1,),
        in_specs=[
            pl.BlockSpec((N, DO), lambda i: (0, 0)),
            pl.BlockSpec((N, DO), lambda i: (0, 0)),
            pl.BlockSpec((MBLK, DO), lambda i: (0, 0)),
            pl.BlockSpec((MBLK, DO), lambda i: (0, 0)),
        ],
        out_specs=pl.BlockSpec((1, 1), lambda i: (0, 0)),
        out_shape=jax.ShapeDtypeStruct((1, 1), F32),
    )(mu, lsr, ps2, ns2)


EPT = E // NS
EPG = E // NW
ACH = 32
ANCH = EPT // ACH


NPT = N // NS
CSR_ACC = (NPT + 1) * DO
DEG_ACC = (NPT + 1) * 16


def _csr_bounds(keys_sorted):
    starts = jnp.searchsorted(keys_sorted, jnp.arange(NS + 1) * NPT)
    s0a = (starts[:NS] // ACH) * ACH
    nch = (starts[1:] - s0a + ACH - 1) // ACH
    b = jnp.stack([s0a, nch], axis=1).reshape(2 * NS).astype(I32)
    return jnp.broadcast_to(b[:, None], (2 * NS, 16)).astype(I32)


def _csragg_body(hwa_h, hwb_h, srcs_h, dsts_h, nrmxs_h, bounds_h, za_h,
                 outa_h, outb_h,
                 bvm, sidxb, didxb, nrmb, rows, acc1, sem):
    cid = lax.axis_index("c")
    sid = lax.axis_index("s")
    lo = sid * NPT
    pltpu.sync_copy(za_h, acc1)
    pltpu.sync_copy(bounds_h.at[pl.ds(sid * 2, 2)], bvm)
    s0 = pl.multiple_of(bvm[0, pl.ds(0, 16)][0], ACH)
    nch = bvm[1, pl.ds(0, 16)][0]

    def work(hw_h, out_h):
        def chunk(c, _):
            e0 = s0 + c * ACH
            pltpu.sync_copy(srcs_h.at[pl.ds(e0, ACH)], sidxb)
            pltpu.sync_copy(dsts_h.at[pl.ds(e0, ACH)], didxb)
            pltpu.sync_copy(nrmxs_h.at[pl.ds(e0, ACH)], nrmb)
            pltpu.async_copy(hw_h.at[sidxb], rows, sem).wait()
            for h in range(ACH // 16):
                dv = didxb[pl.ds(h * 16, 16)]
                oob = (dv < lo) | (dv >= lo + NPT)
                dloc = jnp.where(oob, NPT, dv - lo)
                for i in range(16):
                    dl = dloc[i]
                    nv = nrmb[h * 16 + i, pl.ds(0, 16)]
                    base = pl.multiple_of(dl * DO, 16)
                    for j in range(DO // 16):
                        sl = pl.ds(base + j * 16, 16)
                        acc1[sl] = acc1[sl] + (
                            rows[h * 16 + i, pl.ds(j * 16, 16)] * nv)
            return _

        lax.fori_loop(0, nch, chunk, None)
        pltpu.sync_copy(acc1.at[pl.ds(0, NPT * DO)],
                        out_h.at[pl.ds(lo * DO, NPT * DO)])

    @pl.when(cid == 0)
    def _wa():
        work(hwa_h, outa_h)

    @pl.when(cid == 1)
    def _wb():
        work(hwb_h, outb_h)


def _sc_aggregate_csr(hwa, hwb, srcs_p, dsts_p, nrmxs_p, bounds, za):
    f = pl.kernel(
        _csragg_body,
        out_type=(jax.ShapeDtypeStruct((N * DO,), F32),
                  jax.ShapeDtypeStruct((N * DO,), F32)),
        mesh=_mesh(),
        scratch_types=[
            pltpu.VMEM((2, 16), I32),
            pltpu.VMEM((ACH,), I32),
            pltpu.VMEM((ACH,), I32),
            pltpu.VMEM((ACH, 16), F32),
            pltpu.VMEM((ACH, DO), F32),
            pltpu.VMEM((CSR_ACC,), F32),
            pltpu.SemaphoreType.DMA,
        ],
    )
    oa, ob = f(hwa, hwb, srcs_p, dsts_p, nrmxs_p, bounds, za)
    return oa.reshape(N, DO), ob.reshape(N, DO)


def _csrdeg_body(ewxd_h, dsts_h, boundsd_h, ewxs_h, srcss_h, boundss_h,
                 zd_h, degd_h, degs_h,
                 bvm, kidxb, ewb, dacc, sem):
    cid = lax.axis_index("c")
    sid = lax.axis_index("s")
    lo = sid * NPT
    pltpu.sync_copy(zd_h, dacc)

    def work(ewv_h, keys_h, bnd_h, out_h):
        pltpu.sync_copy(bnd_h.at[pl.ds(sid * 2, 2)], bvm)
        s0 = pl.multiple_of(bvm[0, pl.ds(0, 16)][0], ACH)
        nch = bvm[1, pl.ds(0, 16)][0]

        def chunk(c, _):
            e0 = s0 + c * ACH
            pltpu.sync_copy(keys_h.at[pl.ds(e0, ACH)], kidxb)
            pltpu.sync_copy(ewv_h.at[pl.ds(e0, ACH)], ewb)
            for h in range(ACH // 16):
                dv = kidxb[pl.ds(h * 16, 16)]
                oob = (dv < lo) | (dv >= lo + NPT)
                dloc = jnp.where(oob, NPT, dv - lo)
                for i in range(16):
                    dl = dloc[i]
                    sl = pl.ds(pl.multiple_of(dl * 16, 16), 16)
                    dacc[sl] = dacc[sl] + ewb[h * 16 + i, pl.ds(0, 16)]
            return _

        lax.fori_loop(0, nch, chunk, None)
        pltpu.sync_copy(dacc.at[pl.ds(0, NPT * 16)],
                        out_h.at[pl.ds(lo * 16, NPT * 16)])

    @pl.when(cid == 0)
    def _wd():
        work(ewxd_h, dsts_h, boundsd_h, degd_h)

    @pl.when(cid == 1)
    def _ws():
        work(ewxs_h, srcss_h, boundss_h, degs_h)


def _sc_degrees_csr(ewxd, dsts_p, boundsd, ewxs, srcss_p, boundss, zdeg):
    f = pl.kernel(
        _csrdeg_body,
        out_type=(jax.ShapeDtypeStruct((N * 16,), F32),
                  jax.ShapeDtypeStruct((N * 16,), F32)),
        mesh=_mesh(),
        scratch_types=[
            pltpu.VMEM((2, 16), I32),
            pltpu.VMEM((ACH,), I32),
            pltpu.VMEM((ACH, 16), F32),
            pltpu.VMEM((DEG_ACC,), F32),
            pltpu.SemaphoreType.DMA,
        ],
    )
    degd, degs = f(ewxd, dsts_p, boundsd, ewxs, srcss_p, boundss, zdeg)
    return degd, degs


def _deggat_kernel_body(degs_h, degd_h, src_h, dst_h, prod_h,
                        sidx1, didx1, srows, drows, prodb, sem):
    cid = lax.axis_index("c")
    sid = lax.axis_index("s")
    wid = sid * NC + cid
    g0 = wid * EPG
    pltpu.sync_copy(src_h.at[pl.ds(g0, EPG)], sidx1)
    pltpu.sync_copy(dst_h.at[pl.ds(g0, EPG)], didx1)

    def gat(c, _):
        off = c * ACH
        pltpu.async_copy(degs_h.at[sidx1.at[pl.ds(off, ACH)]], srows,
                         sem).wait()
        pltpu.async_copy(degd_h.at[didx1.at[pl.ds(off, ACH)]], drows,
                         sem).wait()
        sl = pl.ds(0, 16)
        for i in range(ACH):
            prodb[i, sl] = srows[i, sl] * drows[i, sl]
        pltpu.sync_copy(prodb, prod_h.at[pl.ds(g0 + off, ACH)])
        return _

    lax.fori_loop(0, EPG // ACH, gat, None)


def _sc_deg_gather(degs128, degd128, src, dst):
    f2 = pl.kernel(
        _deggat_kernel_body,
        out_type=jax.ShapeDtypeStruct((E, 16), F32),
        mesh=_mesh(),
        scratch_types=[
            pltpu.VMEM((EPG,), I32),
            pltpu.VMEM((EPG,), I32),
            pltpu.VMEM((ACH, 128), F32),
            pltpu.VMEM((ACH, 128), F32),
            pltpu.VMEM((ACH, 16), F32),
            pltpu.SemaphoreType.DMA,
        ],
    )
    return f2(degs128, degd128, src, dst)


PCH = 32


PNCH = (N // NW) // PCH


def _perm_kernel_body(y_h, perm2_h, out_h, idx2, rows, sem):
    cid = lax.axis_index("c")
    sid = lax.axis_index("s")
    wid = sid * NC + cid
    r0 = wid * (N // NW)
    pltpu.sync_copy(perm2_h.at[pl.ds(wid * PNCH, PNCH)], idx2)
    for b in range(PNCH):
        pltpu.async_copy(y_h.at[idx2.at[b]], rows, sem).wait()
        pltpu.sync_copy(rows, out_h.at[pl.ds(r0 + b * PCH, PCH)])


def _sc_perm_gather(y, perm):
    perm2 = perm.reshape(N // PCH, PCH)
    f = pl.kernel(
        _perm_kernel_body,
        out_type=jax.ShapeDtypeStruct((N, DHID), F32),
        mesh=_mesh(),
        scratch_types=[
            pltpu.VMEM((PNCH, PCH), I32),
            pltpu.VMEM((PCH, DHID), F32),
            pltpu.SemaphoreType.DMA,
        ],
    )
    return f(y, perm2)


RPT = N // NS


NSPLIT = 2
DHALF = DO // NSPLIT
NPIECE = DHALF // 16
ACCROWS = N * NPIECE
SRPT = ACCROWS // NS


def _agg_kernel_body(hwa0_h, hwa1_h, hwb0_h, hwb1_h, src_h, dst_h, nrmx_h,
                     z_h, oa0_h, oa1_h, ob0_h, ob1_h,
                     sidx, didx, nrmb, rows, idxb, pieceb, acc_sh, sem):
    cid = lax.axis_index("c")
    sid = lax.axis_index("s")
    b0 = sid * EPT
    pltpu.sync_copy(src_h.at[pl.ds(b0, EPT)], sidx)
    pltpu.sync_copy(dst_h.at[pl.ds(b0, EPT)], didx)

    def one_pass(hw_h, out_h):
        srb = sid * SRPT
        pltpu.sync_copy(z_h.at[pl.ds(srb, SRPT)], acc_sh.at[pl.ds(srb, SRPT)])
        plsc.subcore_barrier()

        def chunk(c, _):
            off = c * ACH
            pltpu.sync_copy(nrmx_h.at[pl.ds(b0 + off, ACH)], nrmb)
            pltpu.async_copy(hw_h.at[sidx.at[pl.ds(off, ACH)]], rows,
                             sem).wait()
            for h in range(ACH // 16):
                hv = didx[pl.ds(off + h * 16, 16)] * NPIECE
                for j in range(NPIECE):
                    idxb[j, pl.ds(h * 16, 16)] = hv + j
            for i in range(ACH):
                nv = nrmb[i, pl.ds(0, 16)]
                for j in range(NPIECE):
                    pieceb[j, i, pl.ds(0, 16)] = rows[i, pl.ds(j * 16, 16)] * nv
            descs = [
                pltpu.async_copy(pieceb.at[j], acc_sh.at[idxb.at[j]], sem,
                                 add=True)
                for j in range(NPIECE)
            ]
            for d in descs:
                d.wait()
            return _

        lax.fori_loop(0, ANCH, chunk, None)
        plsc.subcore_barrier()
        pltpu.sync_copy(acc_sh.at[pl.ds(srb, SRPT)], out_h.at[pl.ds(srb, SRPT)])
        plsc.subcore_barrier()

    @pl.when(cid == 0)
    def _wa():
        one_pass(hwa0_h, oa0_h)
        one_pass(hwa1_h, oa1_h)

    @pl.when(cid == 1)
    def _wb():
        one_pass(hwb0_h, ob0_h)
        one_pass(hwb1_h, ob1_h)


SCH = 64
SNCH = EPG // SCH


def _score_kernel_body(px_h, src_h, dst_h, nsrc_h, ndst_h, ps_h, ns_h,
                       sv, dv, arows, brows, sb, sem):
    cid = lax.axis_index("c")
    sid = lax.axis_index("s")
    wid = sid * NC + cid
    g0 = wid * EPG

    def one_pass(a_h, b_h, out_h):
        pltpu.sync_copy(a_h.at[pl.ds(g0, EPG)], sv)
        pltpu.sync_copy(b_h.at[pl.ds(g0, EPG)], dv)

        def gchunk(c, _):
            off = c * ACH
            pltpu.async_copy(px_h.at[sv.at[pl.ds(off, ACH)]], arows,
                             sem).wait()
            pltpu.async_copy(px_h.at[dv.at[pl.ds(off, ACH)]], brows,
                             sem).wait()
            for i in range(ACH):
                acc = arows[i, pl.ds(0, 16)] * brows[i, pl.ds(0, 16)]
                for j in range(1, DO // 16):
                    sl = pl.ds(j * 16, 16)
                    acc = acc + arows[i, sl] * brows[i, sl]
                sb[i, pl.ds(0, 16)] = acc
            pltpu.sync_copy(sb, out_h.at[pl.ds(g0 + off, ACH)])
            return _

        lax.fori_loop(0, EPG // ACH, gchunk, None)

    one_pass(src_h, dst_h, ps_h)
    one_pass(nsrc_h, ndst_h, ns_h)


def _sc_scores(px, src, dst, nsrc, ndst):
    f = pl.kernel(
        _score_kernel_body,
        out_type=(jax.ShapeDtypeStruct((E, 16), F32),
                  jax.ShapeDtypeStruct((E, 16), F32)),
        mesh=_mesh(),
        scratch_types=[
            pltpu.VMEM((EPG,), I32),
            pltpu.VMEM((EPG,), I32),
            pltpu.VMEM((ACH, DO), F32),
            pltpu.VMEM((ACH, DO), F32),
            pltpu.VMEM((ACH, 16), F32),
            pltpu.SemaphoreType.DMA,
        ],
    )
    ps16, ns16 = f(px, src, dst, nsrc, ndst)
    return jnp.sum(ps16, axis=1), jnp.sum(ns16, axis=1)


_USE_SC_DEG = True
_USE_SC_DEGGAT = True
_USE_SC_PERM = True
_USE_SC_AGG = True
_USE_SC_SCORE = True

def kernel(x, edge_weight, neigh_mask, W_r, b_r, dom_r, W_ri, b_ri, W_r1,
           b_r1, W_r2, b_r2, dom_res, W_g1, W_mu, W_ls, W_t0, b_t0, W_t1,
           b_t1, dom_tg, W_l1, b_l1, dom_1, W_l2, b_l2, edge_index,
           domain_idx):
    src = edge_index[0].astype(I32)
    dst = edge_index[1].astype(I32)

    base = jax.random.key(42)
    eps_p = jax.random.normal(jax.random.fold_in(base, 1), (N, DO), dtype=F32)
    eps_n = jax.random.normal(jax.random.fold_in(base, 6), (N, DO), dtype=F32)
    nsrc = jax.random.randint(jax.random.fold_in(base, 3), (E,), 0, N)
    ndst = jax.random.randint(jax.random.fold_in(base, 4), (E,), 0, N)
    perm = jax.random.permutation(jax.random.fold_in(base, 5), N)

    oh = (domain_idx[:, None] == jnp.arange(4)[None, :]).astype(F32)
    oh2 = jnp.concatenate([oh, oh], axis=0)

    ewx = jnp.broadcast_to(edge_weight[:, None], (E, 16))
    orderd = jnp.argsort(dst)
    orders = jnp.argsort(src)
    srcs_d = src[orderd]
    dsts_d = dst[orderd]
    srcss = src[orders]
    PAD = 64
    zpad_i = jnp.zeros((PAD,), I32)
    npad_i = jnp.full((PAD,), N, I32)
    zpad_f = jnp.zeros((PAD, 16), F32)
    srcs_dp = jnp.concatenate([srcs_d, zpad_i])
    dsts_dp = jnp.concatenate([dsts_d, npad_i])
    srcss_p = jnp.concatenate([srcss, npad_i])
    boundsd = _csr_bounds(dsts_d)
    boundss = _csr_bounds(srcss)

    if _USE_SC_DEG:
        ewxd = jnp.concatenate([ewx[orderd], zpad_f])
        ewxs = jnp.concatenate([ewx[orders], zpad_f])
        zdeg = jnp.zeros((DEG_ACC,), F32)
        degd1, degs1 = _sc_degrees_csr(ewxd, dsts_dp, boundsd, ewxs,
                                       srcss_p, boundss, zdeg)
        degs16 = degs1.reshape(N, 16)
        degd16 = degd1.reshape(N, 16)
        if _USE_SC_DEGGAT:
            degs128 = jnp.pad(degs16, ((0, 0), (0, 112)))
            degd128 = jnp.pad(degd16, ((0, 0), (0, 112)))
            prodx = _sc_deg_gather(degs128, degd128, src, dst)
        else:
            prodx = jnp.broadcast_to(
                (degs16[:, 0][src] * degd16[:, 0][dst])[:, None], (E, 16))
    else:
        deg_s = jax.ops.segment_sum(edge_weight, src, num_segments=N)
        deg_d = jax.ops.segment_sum(edge_weight, dst, num_segments=N)
        prodx = jnp.broadcast_to((deg_s[src] * deg_d[dst])[:, None], (E, 16))
    nrmx = edge_weight[:, None] * lax.rsqrt(jnp.maximum(prodx, 1e-6))
    nrmxs_p = jnp.concatenate([nrmx[orderd], zpad_f])
    zagg = jnp.zeros((CSR_ACC,), F32)

    y = _tc_front(x, W_r, b_r.reshape(1, DHID))
    if _USE_SC_PERM:
        yp = _sc_perm_gather(y, perm.astype(I32))
    else:
        yp = y[perm]
    y_all = jnp.concatenate([y, yp], axis=0)

    feat_x, hw1 = _tc_mid(y_all, oh2, dom_r, W_ri, b_ri.reshape(1, DO),
                          dom_res, W_r1, b_r1.reshape(1, DO), W_r2,
                          b_r2.reshape(1, DO), W_g1)

    def _agg(hwa, hwb):
        if _USE_SC_AGG:
            return _sc_aggregate_csr(hwa, hwb, srcs_dp, dsts_dp, nrmxs_p,
                                     boundsd, zagg)
        nr = nrmx[:, 0]
        outa = jax.ops.segment_sum(hwa[src] * nr[:, None], dst,
                                   num_segments=N)
        outb = jax.ops.segment_sum(hwb[src] * nr[:, None], dst,
                                   num_segments=N)
        return outa, outb

    agg1_p, agg1_n = _agg(hw1[:N], hw1[N:])

    wml = jnp.concatenate([W_mu, W_ls], axis=1)
    hml = _tc_gcnmid(jnp.concatenate([agg1_p, agg1_n], axis=0), wml)

    mu_p, ls_p = _agg(hml[:N, :DO], hml[:N, DO:])
    mu_n, ls_n = _agg(hml[N:, :DO], hml[N:, DO:])

    mu_all = jnp.concatenate([mu_p, mu_n], axis=0)
    ls_all = jnp.concatenate([ls_p, ls_n], axis=0)
    eps_all = jnp.concatenate([eps_p, eps_n], axis=0)

    l2 = _tc_tail(feat_x, mu_all, ls_all, eps_all, oh2,
                  W_t0, b_t0.reshape(1, DO), W_t1, b_t1.reshape(1, DO),
                  dom_tg, W_l1, b_l1.reshape(1, DO), dom_1, W_l2,
                  b_l2.reshape(1, DO))
    pos_x = l2[:N]
    neg_x = l2[N:]

    if _USE_SC_SCORE:
        ps, ns_ = _sc_scores(pos_x, src, dst, nsrc, ndst)
    else:
        gf = _tc_gram(pos_x).reshape(N * N)
        ps = gf[src * N + dst]
        ns_ = gf[nsrc * N + ndst]

    pos_summary = _tc_readout(neigh_mask, pos_x)

    gl = _tc_loss(mu_p, ls_p, ps.reshape(MBLK, DO), ns_.reshape(MBLK, DO))
    graph_loss = gl[0, 0]

    return pos_x, neg_x, pos_summary, graph_loss


# speedup vs baseline: 1.4895x; 1.1387x over previous
import functools

import jax
import jax.numpy as jnp
from jax import lax
from jax.experimental import pallas as pl
from jax.experimental.pallas import tpu as pltpu
from jax.experimental.pallas import tpu_sc as plsc

N = 4096
E = 131072
DIN = 512
DHID = 1024
DO = 256

F32 = jnp.float32
I32 = jnp.int32

NC = 2
NS = 16
NW = NC * NS

_SC_MESH = dict(core_axis_name="c", subcore_axis_name="s", num_cores=NC,
                num_subcores=NS)


def _mesh():
    return plsc.VectorSubcoreMesh(**_SC_MESH)


MBLK = 512


def _front_body(x_ref, w_ref, b_ref, y_ref):
    y_ref[...] = jnp.dot(x_ref[...], w_ref[...],
                         preferred_element_type=F32) + b_ref[...]


def _tc_front(x, w, b2):
    return pl.pallas_call(
        _front_body,
        grid=(N // MBLK,),
        in_specs=[
            pl.BlockSpec((MBLK, DIN), lambda i: (i, 0)),
            pl.BlockSpec((DIN, DHID), lambda i: (0, 0)),
            pl.BlockSpec((1, DHID), lambda i: (0, 0)),
        ],
        out_specs=pl.BlockSpec((MBLK, DHID), lambda i: (i, 0)),
        out_shape=jax.ShapeDtypeStruct((N, DHID), F32),
    )(x, w, b2)


def _mid_body(y_ref, oh_ref, domr_ref, wri_ref, bri_ref, domres_ref,
              wr1_ref, br1_ref, wr2_ref, br2_ref, wg1_ref,
              feat_ref, hw1_ref):
    oh = oh_ref[...]
    h1 = jnp.maximum(y_ref[...], 0.0) + jnp.dot(
        oh, domr_ref[...], preferred_element_type=F32)
    h2 = jnp.maximum(
        jnp.dot(h1, wri_ref[...], preferred_element_type=F32) + bri_ref[...],
        0.0) + jnp.dot(oh, domres_ref[...], preferred_element_type=F32)
    h3 = h2 + jnp.maximum(
        jnp.dot(h2, wr1_ref[...], preferred_element_type=F32) + br1_ref[...],
        0.0)
    h4 = h3 + jnp.maximum(
        jnp.dot(h3, wr2_ref[...], preferred_element_type=F32) + br2_ref[...],
        0.0)
    feat_ref[...] = h4
    hw1_ref[...] = jnp.dot(h4, wg1_ref[...], preferred_element_type=F32)


def _tc_mid(y_all, oh2, dom_r, w_ri, bri2, dom_res, w_r1, br12, w_r2, br22,
            w_g1):
    m = y_all.shape[0]
    return pl.pallas_call(
        _mid_body,
        grid=(m // MBLK,),
        in_specs=[
            pl.BlockSpec((MBLK, DHID), lambda i: (i, 0)),
            pl.BlockSpec((MBLK, 4), lambda i: (i, 0)),
            pl.BlockSpec((4, DHID), lambda i: (0, 0)),
            pl.BlockSpec((DHID, DO), lambda i: (0, 0)),
            pl.BlockSpec((1, DO), lambda i: (0, 0)),
            pl.BlockSpec((4, DO), lambda i: (0, 0)),
            pl.BlockSpec((DO, DO), lambda i: (0, 0)),
            pl.BlockSpec((1, DO), lambda i: (0, 0)),
            pl.BlockSpec((DO, DO), lambda i: (0, 0)),
            pl.BlockSpec((1, DO), lambda i: (0, 0)),
            pl.BlockSpec((DO, DO), lambda i: (0, 0)),
        ],
        out_specs=[
            pl.BlockSpec((MBLK, DO), lambda i: (i, 0)),
            pl.BlockSpec((MBLK, DO), lambda i: (i, 0)),
        ],
        out_shape=[
            jax.ShapeDtypeStruct((m, DO), F32),
            jax.ShapeDtypeStruct((m, DO), F32),
        ],
    )(y_all, oh2, dom_r, w_ri, bri2, dom_res, w_r1, br12, w_r2, br22, w_g1)


def _gcnmid_body(agg_ref, wml_ref, out_ref):
    hid = jnp.maximum(agg_ref[...], 0.0)
    out_ref[...] = jnp.dot(hid, wml_ref[...], preferred_element_type=F32)


def _tc_gcnmid(agg_all, wml):
    m = agg_all.shape[0]
    return pl.pallas_call(
        _gcnmid_body,
        grid=(m // MBLK,),
        in_specs=[
            pl.BlockSpec((MBLK, DO), lambda i: (i, 0)),
            pl.BlockSpec((DO, 2 * DO), lambda i: (0, 0)),
        ],
        out_specs=pl.BlockSpec((MBLK, 2 * DO), lambda i: (i, 0)),
        out_shape=jax.ShapeDtypeStruct((m, 2 * DO), F32),
    )(agg_all, wml)


def _tail_body(feat_ref, mu_ref, lsr_ref, eps_ref, oh_ref,
               wt0_ref, bt0_ref, wt1_ref, bt1_ref, domtg_ref,
               wl1_ref, bl1_ref, dom1_ref, wl2_ref, bl2_ref, out_ref):
    oh = oh_ref[...]
    ls = jnp.clip(lsr_ref[...], -10.0, 10.0)
    z = mu_ref[...] + eps_ref[...] * jnp.exp(ls)
    g = jnp.maximum(
        jnp.dot(z, wt0_ref[...], preferred_element_type=F32) + bt0_ref[...],
        0.0)
    g = jnp.maximum(
        jnp.dot(g, wt1_ref[...], preferred_element_type=F32) + bt1_ref[...],
        0.0) + jnp.dot(oh, domtg_ref[...], preferred_element_type=F32)
    feat = jnp.concatenate([feat_ref[...], g], axis=1)
    l1 = jnp.maximum(
        jnp.dot(feat, wl1_ref[...], preferred_element_type=F32) + bl1_ref[...],
        0.0) + jnp.dot(oh, dom1_ref[...], preferred_element_type=F32)
    out_ref[...] = jnp.maximum(
        jnp.dot(l1, wl2_ref[...], preferred_element_type=F32) + bl2_ref[...],
        0.0)


def _tc_tail(feat, mu, lsr, eps, oh2, w_t0, bt02, w_t1, bt12, dom_tg,
             w_l1, bl12, dom_1, w_l2, bl22):
    m = feat.shape[0]
    return pl.pallas_call(
        _tail_body,
        grid=(m // MBLK,),
        in_specs=[
            pl.BlockSpec((MBLK, DO), lambda i: (i, 0)),
            pl.BlockSpec((MBLK, DO), lambda i: (i, 0)),
            pl.BlockSpec((MBLK, DO), lambda i: (i, 0)),
            pl.BlockSpec((MBLK, DO), lambda i: (i, 0)),
            pl.BlockSpec((MBLK, 4), lambda i: (i, 0)),
            pl.BlockSpec((DO, DO), lambda i: (0, 0)),
            pl.BlockSpec((1, DO), lambda i: (0, 0)),
            pl.BlockSpec((DO, DO), lambda i: (0, 0)),
            pl.BlockSpec((1, DO), lambda i: (0, 0)),
            pl.BlockSpec((4, DO), lambda i: (0, 0)),
            pl.BlockSpec((2 * DO, DO), lambda i: (0, 0)),
            pl.BlockSpec((1, DO), lambda i: (0, 0)),
            pl.BlockSpec((4, DO), lambda i: (0, 0)),
            pl.BlockSpec((DO, DO), lambda i: (0, 0)),
            pl.BlockSpec((1, DO), lambda i: (0, 0)),
        ],
        out_specs=pl.BlockSpec((MBLK, DO), lambda i: (i, 0)),
        out_shape=jax.ShapeDtypeStruct((m, DO), F32),
    )(feat, mu, lsr, eps, oh2, w_t0, bt02, w_t1, bt12, dom_tg,
      w_l1, bl12, dom_1, w_l2, bl22)


def _gram_body(pxb_ref, px_ref, g_ref):
    g_ref[...] = lax.dot_general(
        pxb_ref[...], px_ref[...], (((1,), (1,)), ((), ())),
        preferred_element_type=F32)


def _tc_gram(px):
    return pl.pallas_call(
        _gram_body,
        grid=(N // MBLK,),
        in_specs=[
            pl.BlockSpec((MBLK, DO), lambda i: (i, 0)),
            pl.BlockSpec((N, DO), lambda i: (0, 0)),
        ],
        out_specs=pl.BlockSpec((MBLK, N), lambda i: (i, 0)),
        out_shape=jax.ShapeDtypeStruct((N, N), F32),
    )(px, px)


def _readout_body(nm_ref, px_ref, out_ref):
    nm = nm_ref[...]
    vsum = jnp.dot(nm, px_ref[...], preferred_element_type=F32)
    rsum = jnp.sum(nm, axis=1, keepdims=True)
    gz = vsum / rsum
    nrm = jnp.sqrt(jnp.sum(gz * gz, axis=1, keepdims=True))
    out_ref[...] = gz / jnp.maximum(nrm, 1e-12)


def _tc_readout(nm, px):
    return pl.pallas_call(
        _readout_body,
        grid=(N // MBLK,),
        in_specs=[
            pl.BlockSpec((MBLK, N), lambda i: (i, 0)),
            pl.BlockSpec((N, DO), lambda i: (0, 0)),
        ],
        out_specs=pl.BlockSpec((MBLK, DO), lambda i: (i, 0)),
        out_shape=jax.ShapeDtypeStruct((N, DO), F32),
    )(nm, px)


def _loss_body(mu_ref, lsr_ref, ps_ref, ns_ref, out_ref):
    mu = mu_ref[...]
    ls = jnp.clip(lsr_ref[...], -10.0, 10.0)
    t = 1.0 + 2.0 * ls - mu * mu - jnp.exp(2.0 * ls)
    kl = -0.5 * (jnp.sum(t) / N)
    ps = ps_ref[...]
    ns = ns_ref[...]
    pterm = jnp.log(jnp.maximum(jax.nn.sigmoid(ps), 1e-15))
    nterm = jnp.log(jnp.maximum(1.0 - jax.nn.sigmoid(ns), 1e-15))
    recon = -(jnp.sum(pterm) / E) - (jnp.sum(nterm) / E)
    out_ref[...] = jnp.reshape(kl / N + recon, (1, 1))


def _tc_loss(mu, lsr, ps2, ns2):
    return pl.pallas_call(
        _loss_body,
        grid=(1,),
        in_specs=[
            pl.BlockSpec((N, DO), lambda i: (0, 0)),
            pl.BlockSpec((N, DO), lambda i: (0, 0)),
            pl.BlockSpec((MBLK, DO), lambda i: (0, 0)),
            pl.BlockSpec((MBLK, DO), lambda i: (0, 0)),
        ],
        out_specs=pl.BlockSpec((1, 1), lambda i: (0, 0)),
        out_shape=jax.ShapeDtypeStruct((1, 1), F32),
    )(mu, lsr, ps2, ns2)


EPT = E // NS
EPG = E // NW
ACH = 32
ANCH = EPT // ACH


NPT = N // NS
CSR_ACC = (NPT + 1) * DO
DEG_ACC = (NPT + 1) * 16


def _csr_bounds(keys_sorted):
    starts = jnp.searchsorted(keys_sorted, jnp.arange(NS + 1) * NPT)
    s0a = (starts[:NS] // ACH) * ACH
    nch = (starts[1:] - s0a + ACH - 1) // ACH
    b = jnp.stack([s0a, nch], axis=1).reshape(2 * NS).astype(I32)
    return jnp.broadcast_to(b[:, None], (2 * NS, 16)).astype(I32)


def _csragg_body(hwa_h, hwb_h, srcs_h, dsts_h, nrmxs_h, bounds_h, za_h,
                 outa_h, outb_h,
                 bvm, sidxb, didxb, nrmb, rows, acc1, sem):
    cid = lax.axis_index("c")
    sid = lax.axis_index("s")
    lo = sid * NPT
    pltpu.sync_copy(za_h, acc1)
    pltpu.sync_copy(bounds_h.at[pl.ds(sid * 2, 2)], bvm)
    s0 = pl.multiple_of(bvm[0, pl.ds(0, 16)][0], ACH)
    nch = bvm[1, pl.ds(0, 16)][0]

    def work(hw_h, out_h):
        def chunk(c, _):
            e0 = s0 + c * ACH
            d1 = pltpu.async_copy(srcs_h.at[pl.ds(e0, ACH)], sidxb, sem)
            d2 = pltpu.async_copy(dsts_h.at[pl.ds(e0, ACH)], didxb, sem)
            d3 = pltpu.async_copy(nrmxs_h.at[pl.ds(e0, ACH)], nrmb, sem)
            d1.wait()
            d4 = pltpu.async_copy(hw_h.at[sidxb], rows, sem)
            d2.wait()
            d3.wait()
            d4.wait()
            for h in range(ACH // 16):
                dv = didxb[pl.ds(h * 16, 16)]
                oob = (dv < lo) | (dv >= lo + NPT)
                dloc = jnp.where(oob, NPT, dv - lo)
                for i in range(16):
                    dl = dloc[i]
                    nv = nrmb[h * 16 + i, pl.ds(0, 16)]
                    base = pl.multiple_of(dl * DO, 16)
                    for j in range(DO // 16):
                        sl = pl.ds(base + j * 16, 16)
                        acc1[sl] = acc1[sl] + (
                            rows[h * 16 + i, pl.ds(j * 16, 16)] * nv)
            return _

        lax.fori_loop(0, nch, chunk, None)
        pltpu.sync_copy(acc1.at[pl.ds(0, NPT * DO)],
                        out_h.at[pl.ds(lo * DO, NPT * DO)])

    @pl.when(cid == 0)
    def _wa():
        work(hwa_h, outa_h)

    @pl.when(cid == 1)
    def _wb():
        work(hwb_h, outb_h)


def _sc_aggregate_csr(hwa, hwb, srcs_p, dsts_p, nrmxs_p, bounds, za):
    f = pl.kernel(
        _csragg_body,
        out_type=(jax.ShapeDtypeStruct((N * DO,), F32),
                  jax.ShapeDtypeStruct((N * DO,), F32)),
        mesh=_mesh(),
        scratch_types=[
            pltpu.VMEM((2, 16), I32),
            pltpu.VMEM((ACH,), I32),
            pltpu.VMEM((ACH,), I32),
            pltpu.VMEM((ACH, 16), F32),
            pltpu.VMEM((ACH, DO), F32),
            pltpu.VMEM((CSR_ACC,), F32),
            pltpu.SemaphoreType.DMA,
        ],
    )
    oa, ob = f(hwa, hwb, srcs_p, dsts_p, nrmxs_p, bounds, za)
    return oa.reshape(N, DO), ob.reshape(N, DO)


def _csrdeg_body(ewxd_h, dsts_h, boundsd_h, ewxs_h, srcss_h, boundss_h,
                 zd_h, degd_h, degs_h,
                 bvm, kidxb, ewb, dacc, sem):
    cid = lax.axis_index("c")
    sid = lax.axis_index("s")
    lo = sid * NPT
    pltpu.sync_copy(zd_h, dacc)

    def work(ewv_h, keys_h, bnd_h, out_h):
        pltpu.sync_copy(bnd_h.at[pl.ds(sid * 2, 2)], bvm)
        s0 = pl.multiple_of(bvm[0, pl.ds(0, 16)][0], ACH)
        nch = bvm[1, pl.ds(0, 16)][0]

        def chunk(c, _):
            e0 = s0 + c * ACH
            pltpu.sync_copy(keys_h.at[pl.ds(e0, ACH)], kidxb)
            pltpu.sync_copy(ewv_h.at[pl.ds(e0, ACH)], ewb)
            for h in range(ACH // 16):
                dv = kidxb[pl.ds(h * 16, 16)]
                oob = (dv < lo) | (dv >= lo + NPT)
                dloc = jnp.where(oob, NPT, dv - lo)
                for i in range(16):
                    dl = dloc[i]
                    sl = pl.ds(pl.multiple_of(dl * 16, 16), 16)
                    dacc[sl] = dacc[sl] + ewb[h * 16 + i, pl.ds(0, 16)]
            return _

        lax.fori_loop(0, nch, chunk, None)
        pltpu.sync_copy(dacc.at[pl.ds(0, NPT * 16)],
                        out_h.at[pl.ds(lo * 16, NPT * 16)])

    @pl.when(cid == 0)
    def _wd():
        work(ewxd_h, dsts_h, boundsd_h, degd_h)

    @pl.when(cid == 1)
    def _ws():
        work(ewxs_h, srcss_h, boundss_h, degs_h)


def _sc_degrees_csr(ewxd, dsts_p, boundsd, ewxs, srcss_p, boundss, zdeg):
    f = pl.kernel(
        _csrdeg_body,
        out_type=(jax.ShapeDtypeStruct((N * 16,), F32),
                  jax.ShapeDtypeStruct((N * 16,), F32)),
        mesh=_mesh(),
        scratch_types=[
            pltpu.VMEM((2, 16), I32),
            pltpu.VMEM((ACH,), I32),
            pltpu.VMEM((ACH, 16), F32),
            pltpu.VMEM((DEG_ACC,), F32),
            pltpu.SemaphoreType.DMA,
        ],
    )
    degd, degs = f(ewxd, dsts_p, boundsd, ewxs, srcss_p, boundss, zdeg)
    return degd, degs


def _deggat_kernel_body(degs_h, degd_h, src_h, dst_h, prod_h,
                        sidx1, didx1, srows, drows, prodb, sem):
    cid = lax.axis_index("c")
    sid = lax.axis_index("s")
    wid = sid * NC + cid
    g0 = wid * EPG
    pltpu.sync_copy(src_h.at[pl.ds(g0, EPG)], sidx1)
    pltpu.sync_copy(dst_h.at[pl.ds(g0, EPG)], didx1)

    def gat(c, _):
        off = c * ACH
        d1 = pltpu.async_copy(degs_h.at[sidx1.at[pl.ds(off, ACH)]], srows,
                              sem)
        d2 = pltpu.async_copy(degd_h.at[didx1.at[pl.ds(off, ACH)]], drows,
                              sem)
        d1.wait()
        d2.wait()
        sl = pl.ds(0, 16)
        for i in range(ACH):
            prodb[i, sl] = srows[i, sl] * drows[i, sl]
        pltpu.sync_copy(prodb, prod_h.at[pl.ds(g0 + off, ACH)])
        return _

    lax.fori_loop(0, EPG // ACH, gat, None)


def _sc_deg_gather(degs128, degd128, src, dst):
    f2 = pl.kernel(
        _deggat_kernel_body,
        out_type=jax.ShapeDtypeStruct((E, 16), F32),
        mesh=_mesh(),
        scratch_types=[
            pltpu.VMEM((EPG,), I32),
            pltpu.VMEM((EPG,), I32),
            pltpu.VMEM((ACH, 128), F32),
            pltpu.VMEM((ACH, 128), F32),
            pltpu.VMEM((ACH, 16), F32),
            pltpu.SemaphoreType.DMA,
        ],
    )
    return f2(degs128, degd128, src, dst)


PCH = 32


PNCH = (N // NW) // PCH


def _perm_kernel_body(y_h, perm2_h, out_h, idx2, rows, sem):
    cid = lax.axis_index("c")
    sid = lax.axis_index("s")
    wid = sid * NC + cid
    r0 = wid * (N // NW)
    pltpu.sync_copy(perm2_h.at[pl.ds(wid * PNCH, PNCH)], idx2)
    for b in range(PNCH):
        pltpu.async_copy(y_h.at[idx2.at[b]], rows, sem).wait()
        pltpu.sync_copy(rows, out_h.at[pl.ds(r0 + b * PCH, PCH)])


def _sc_perm_gather(y, perm):
    perm2 = perm.reshape(N // PCH, PCH)
    f = pl.kernel(
        _perm_kernel_body,
        out_type=jax.ShapeDtypeStruct((N, DHID), F32),
        mesh=_mesh(),
        scratch_types=[
            pltpu.VMEM((PNCH, PCH), I32),
            pltpu.VMEM((PCH, DHID), F32),
            pltpu.SemaphoreType.DMA,
        ],
    )
    return f(y, perm2)


RPT = N // NS


NSPLIT = 2
DHALF = DO // NSPLIT
NPIECE = DHALF // 16
ACCROWS = N * NPIECE
SRPT = ACCROWS // NS


def _agg_kernel_body(hwa0_h, hwa1_h, hwb0_h, hwb1_h, src_h, dst_h, nrmx_h,
                     z_h, oa0_h, oa1_h, ob0_h, ob1_h,
                     sidx, didx, nrmb, rows, idxb, pieceb, acc_sh, sem):
    cid = lax.axis_index("c")
    sid = lax.axis_index("s")
    b0 = sid * EPT
    pltpu.sync_copy(src_h.at[pl.ds(b0, EPT)], sidx)
    pltpu.sync_copy(dst_h.at[pl.ds(b0, EPT)], didx)

    def one_pass(hw_h, out_h):
        srb = sid * SRPT
        pltpu.sync_copy(z_h.at[pl.ds(srb, SRPT)], acc_sh.at[pl.ds(srb, SRPT)])
        plsc.subcore_barrier()

        def chunk(c, _):
            off = c * ACH
            pltpu.sync_copy(nrmx_h.at[pl.ds(b0 + off, ACH)], nrmb)
            pltpu.async_copy(hw_h.at[sidx.at[pl.ds(off, ACH)]], rows,
                             sem).wait()
            for h in range(ACH // 16):
                hv = didx[pl.ds(off + h * 16, 16)] * NPIECE
                for j in range(NPIECE):
                    idxb[j, pl.ds(h * 16, 16)] = hv + j
            for i in range(ACH):
                nv = nrmb[i, pl.ds(0, 16)]
                for j in range(NPIECE):
                    pieceb[j, i, pl.ds(0, 16)] = rows[i, pl.ds(j * 16, 16)] * nv
            descs = [
                pltpu.async_copy(pieceb.at[j], acc_sh.at[idxb.at[j]], sem,
                                 add=True)
                for j in range(NPIECE)
            ]
            for d in descs:
                d.wait()
            return _

        lax.fori_loop(0, ANCH, chunk, None)
        plsc.subcore_barrier()
        pltpu.sync_copy(acc_sh.at[pl.ds(srb, SRPT)], out_h.at[pl.ds(srb, SRPT)])
        plsc.subcore_barrier()

    @pl.when(cid == 0)
    def _wa():
        one_pass(hwa0_h, oa0_h)
        one_pass(hwa1_h, oa1_h)

    @pl.when(cid == 1)
    def _wb():
        one_pass(hwb0_h, ob0_h)
        one_pass(hwb1_h, ob1_h)


SCH = 64
SNCH = EPG // SCH


def _score_kernel_body(px_h, src_h, dst_h, nsrc_h, ndst_h, ps_h, ns_h,
                       sv, dv, arows, brows, sb, sem):
    cid = lax.axis_index("c")
    sid = lax.axis_index("s")
    wid = sid * NC + cid
    g0 = wid * EPG

    def one_pass(a_h, b_h, out_h):
        pltpu.sync_copy(a_h.at[pl.ds(g0, EPG)], sv)
        pltpu.sync_copy(b_h.at[pl.ds(g0, EPG)], dv)

        def gchunk(c, _):
            off = c * ACH
            d1 = pltpu.async_copy(px_h.at[sv.at[pl.ds(off, ACH)]], arows,
                                  sem)
            d2 = pltpu.async_copy(px_h.at[dv.at[pl.ds(off, ACH)]], brows,
                                  sem)
            d1.wait()
            d2.wait()
            for i in range(ACH):
                acc = arows[i, pl.ds(0, 16)] * brows[i, pl.ds(0, 16)]
                for j in range(1, DO // 16):
                    sl = pl.ds(j * 16, 16)
                    acc = acc + arows[i, sl] * brows[i, sl]
                sb[i, pl.ds(0, 16)] = acc
            pltpu.sync_copy(sb, out_h.at[pl.ds(g0 + off, ACH)])
            return _

        lax.fori_loop(0, EPG // ACH, gchunk, None)

    one_pass(src_h, dst_h, ps_h)
    one_pass(nsrc_h, ndst_h, ns_h)


def _sc_scores(px, src, dst, nsrc, ndst):
    f = pl.kernel(
        _score_kernel_body,
        out_type=(jax.ShapeDtypeStruct((E, 16), F32),
                  jax.ShapeDtypeStruct((E, 16), F32)),
        mesh=_mesh(),
        scratch_types=[
            pltpu.VMEM((EPG,), I32),
            pltpu.VMEM((EPG,), I32),
            pltpu.VMEM((ACH, DO), F32),
            pltpu.VMEM((ACH, DO), F32),
            pltpu.VMEM((ACH, 16), F32),
            pltpu.SemaphoreType.DMA,
        ],
    )
    ps16, ns16 = f(px, src, dst, nsrc, ndst)
    return jnp.sum(ps16, axis=1), jnp.sum(ns16, axis=1)


_USE_SC_DEG = True
_USE_SC_DEGGAT = True
_USE_SC_PERM = True
_USE_SC_AGG = True
_USE_SC_SCORE = True

def kernel(x, edge_weight, neigh_mask, W_r, b_r, dom_r, W_ri, b_ri, W_r1,
           b_r1, W_r2, b_r2, dom_res, W_g1, W_mu, W_ls, W_t0, b_t0, W_t1,
           b_t1, dom_tg, W_l1, b_l1, dom_1, W_l2, b_l2, edge_index,
           domain_idx):
    src = edge_index[0].astype(I32)
    dst = edge_index[1].astype(I32)

    base = jax.random.key(42)
    eps_p = jax.random.normal(jax.random.fold_in(base, 1), (N, DO), dtype=F32)
    eps_n = jax.random.normal(jax.random.fold_in(base, 6), (N, DO), dtype=F32)
    nsrc = jax.random.randint(jax.random.fold_in(base, 3), (E,), 0, N)
    ndst = jax.random.randint(jax.random.fold_in(base, 4), (E,), 0, N)
    perm = jax.random.permutation(jax.random.fold_in(base, 5), N)

    oh = (domain_idx[:, None] == jnp.arange(4)[None, :]).astype(F32)
    oh2 = jnp.concatenate([oh, oh], axis=0)

    ewx = jnp.broadcast_to(edge_weight[:, None], (E, 16))
    orderd = jnp.argsort(dst)
    orders = jnp.argsort(src)
    srcs_d = src[orderd]
    dsts_d = dst[orderd]
    srcss = src[orders]
    PAD = 64
    zpad_i = jnp.zeros((PAD,), I32)
    npad_i = jnp.full((PAD,), N, I32)
    zpad_f = jnp.zeros((PAD, 16), F32)
    srcs_dp = jnp.concatenate([srcs_d, zpad_i])
    dsts_dp = jnp.concatenate([dsts_d, npad_i])
    srcss_p = jnp.concatenate([srcss, npad_i])
    boundsd = _csr_bounds(dsts_d)
    boundss = _csr_bounds(srcss)

    if _USE_SC_DEG:
        ewxd = jnp.concatenate([ewx[orderd], zpad_f])
        ewxs = jnp.concatenate([ewx[orders], zpad_f])
        zdeg = jnp.zeros((DEG_ACC,), F32)
        degd1, degs1 = _sc_degrees_csr(ewxd, dsts_dp, boundsd, ewxs,
                                       srcss_p, boundss, zdeg)
        degs16 = degs1.reshape(N, 16)
        degd16 = degd1.reshape(N, 16)
        if _USE_SC_DEGGAT:
            degs128 = jnp.pad(degs16, ((0, 0), (0, 112)))
            degd128 = jnp.pad(degd16, ((0, 0), (0, 112)))
            prodx = _sc_deg_gather(degs128, degd128, src, dst)
        else:
            prodx = jnp.broadcast_to(
                (degs16[:, 0][src] * degd16[:, 0][dst])[:, None], (E, 16))
    else:
        deg_s = jax.ops.segment_sum(edge_weight, src, num_segments=N)
        deg_d = jax.ops.segment_sum(edge_weight, dst, num_segments=N)
        prodx = jnp.broadcast_to((deg_s[src] * deg_d[dst])[:, None], (E, 16))
    nrmx = edge_weight[:, None] * lax.rsqrt(jnp.maximum(prodx, 1e-6))
    nrmxs_p = jnp.concatenate([nrmx[orderd], zpad_f])
    zagg = jnp.zeros((CSR_ACC,), F32)

    y = _tc_front(x, W_r, b_r.reshape(1, DHID))
    if _USE_SC_PERM:
        yp = _sc_perm_gather(y, perm.astype(I32))
    else:
        yp = y[perm]
    y_all = jnp.concatenate([y, yp], axis=0)

    feat_x, hw1 = _tc_mid(y_all, oh2, dom_r, W_ri, b_ri.reshape(1, DO),
                          dom_res, W_r1, b_r1.reshape(1, DO), W_r2,
                          b_r2.reshape(1, DO), W_g1)

    def _agg(hwa, hwb):
        if _USE_SC_AGG:
            return _sc_aggregate_csr(hwa, hwb, srcs_dp, dsts_dp, nrmxs_p,
                                     boundsd, zagg)
        nr = nrmx[:, 0]
        outa = jax.ops.segment_sum(hwa[src] * nr[:, None], dst,
                                   num_segments=N)
        outb = jax.ops.segment_sum(hwb[src] * nr[:, None], dst,
                                   num_segments=N)
        return outa, outb

    agg1_p, agg1_n = _agg(hw1[:N], hw1[N:])

    wml = jnp.concatenate([W_mu, W_ls], axis=1)
    hml = _tc_gcnmid(jnp.concatenate([agg1_p, agg1_n], axis=0), wml)

    mu_p, ls_p = _agg(hml[:N, :DO], hml[:N, DO:])
    mu_n, ls_n = _agg(hml[N:, :DO], hml[N:, DO:])

    mu_all = jnp.concatenate([mu_p, mu_n], axis=0)
    ls_all = jnp.concatenate([ls_p, ls_n], axis=0)
    eps_all = jnp.concatenate([eps_p, eps_n], axis=0)

    l2 = _tc_tail(feat_x, mu_all, ls_all, eps_all, oh2,
                  W_t0, b_t0.reshape(1, DO), W_t1, b_t1.reshape(1, DO),
                  dom_tg, W_l1, b_l1.reshape(1, DO), dom_1, W_l2,
                  b_l2.reshape(1, DO))
    pos_x = l2[:N]
    neg_x = l2[N:]

    if _USE_SC_SCORE:
        ps, ns_ = _sc_scores(pos_x, src, dst, nsrc, ndst)
    else:
        gf = _tc_gram(pos_x).reshape(N * N)
        ps = gf[src * N + dst]
        ns_ = gf[nsrc * N + ndst]

    pos_summary = _tc_readout(neigh_mask, pos_x)

    gl = _tc_loss(mu_p, ls_p, ps.reshape(MBLK, DO), ns_.reshape(MBLK, DO))
    graph_loss = gl[0, 0]

    return pos_x, neg_x, pos_summary, graph_loss
